# baseline (speedup 1.0000x reference)
import functools

import jax
import jax.numpy as jnp
from jax import lax
from jax.experimental import pallas as pl
from jax.experimental.pallas import tpu as pltpu
from jax.experimental.pallas import tpu_sc as plsc

IMG = 512.0
C_IN = 39
C_OUT = 31
N = 640000
NW = 32
PER_W = N // NW
CHUNK = 800
NCHUNK = PER_W // CHUNK
GROUPS = CHUNK // 16

_mesh = plsc.VectorSubcoreMesh(core_axis_name="c", subcore_axis_name="s")


@functools.partial(
    pl.kernel,
    mesh=_mesh,
    out_type=jax.ShapeDtypeStruct((N, C_OUT), jnp.float32),
    scratch_types=[
        pltpu.VMEM((CHUNK, C_IN), jnp.float32),
        pltpu.VMEM((CHUNK, C_OUT), jnp.float32),
    ],
    compiler_params=pltpu.CompilerParams(
        needs_layout_passes=False, use_tc_tiling_on_sc=False
    ),
)
def _decode_sc(x_hbm, o_hbm, in_v, out_v):
    cid = lax.axis_index("c")
    sid = lax.axis_index("s")
    wid = sid * 2 + cid
    base = wid * PER_W
    lanes = lax.iota(jnp.int32, 16)

    def chunk_body(ci, carry):
        cbase = base + ci * CHUNK
        pltpu.sync_copy(x_hbm.at[pl.ds(cbase, CHUNK), :], in_v)

        def group_body(g, carry2):
            rows = g * 16 + lanes

            def gat(c):
                col = jnp.full((16,), c, jnp.int32)
                return plsc.load_gather(in_v, [rows, col])

            def scat(c, val):
                col = jnp.full((16,), c, jnp.int32)
                plsc.store_scatter(out_v, [rows, col], val)

            cx = gat(31)
            cy = gat(32)
            vxw = gat(35) * gat(33)
            vyh = gat(36) * gat(34)
            for i in range(5):
                kx = (gat(21 + 2 * i) * vxw + cx) * IMG
                ky = (gat(22 + 2 * i) * vyh + cy) * IMG
                scat(21 + 2 * i, kx)
                scat(22 + 2 * i, ky)
            for c in range(21):
                scat(c, gat(c))
            return carry2

        lax.fori_loop(0, GROUPS, group_body, 0)
        pltpu.sync_copy(out_v, o_hbm.at[pl.ds(cbase, CHUNK), :])
        return carry

    lax.fori_loop(0, NCHUNK, chunk_body, 0)


@jax.jit
def kernel(y_pred):
    bt, nb, _ = y_pred.shape
    x = y_pred.reshape(bt * nb, C_IN)
    out = _decode_sc(x)
    return out.reshape(bt, nb, C_OUT)


# speedup vs baseline: 1.1125x; 1.1125x over previous
import functools

import jax
import jax.numpy as jnp
from jax import lax
from jax.experimental import pallas as pl
from jax.experimental.pallas import tpu as pltpu
from jax.experimental.pallas import tpu_sc as plsc

IMG = 512.0
C_IN = 39
C_OUT = 31
N = 640000
NW = 32
PER_W = N // NW
CHUNK = 160
NCHUNK = PER_W // CHUNK
GROUPS = CHUNK // 16

_mesh = plsc.VectorSubcoreMesh(core_axis_name="c", subcore_axis_name="s")


@functools.partial(
    pl.kernel,
    mesh=_mesh,
    out_type=jax.ShapeDtypeStruct((N, C_OUT), jnp.float32),
    scratch_types=[
        pltpu.VMEM((CHUNK, C_IN), jnp.float32),
        pltpu.VMEM((CHUNK, C_OUT), jnp.float32),
    ],
    compiler_params=pltpu.CompilerParams(
        needs_layout_passes=False, use_tc_tiling_on_sc=True
    ),
)
def _decode_sc(x_hbm, o_hbm, in_v, out_v):
    cid = lax.axis_index("c")
    sid = lax.axis_index("s")
    wid = sid * 2 + cid
    base = wid * PER_W
    lanes = lax.iota(jnp.int32, 16)

    def chunk_body(ci, carry):
        cbase = base + ci * CHUNK
        pltpu.sync_copy(x_hbm.at[pl.ds(cbase, CHUNK), :], in_v)

        def group_body(g, carry2):
            rows = g * 16 + lanes

            def gat(c):
                col = jnp.full((16,), c, jnp.int32)
                return plsc.load_gather(in_v, [rows, col])

            def scat(c, val):
                col = jnp.full((16,), c, jnp.int32)
                plsc.store_scatter(out_v, [rows, col], val)

            cx = gat(31)
            cy = gat(32)
            vxw = gat(35) * gat(33)
            vyh = gat(36) * gat(34)
            for i in range(5):
                kx = (gat(21 + 2 * i) * vxw + cx) * IMG
                ky = (gat(22 + 2 * i) * vyh + cy) * IMG
                scat(21 + 2 * i, kx)
                scat(22 + 2 * i, ky)
            for c in range(21):
                scat(c, gat(c))
            return carry2

        lax.fori_loop(0, GROUPS, group_body, 0)
        pltpu.sync_copy(out_v, o_hbm.at[pl.ds(cbase, CHUNK), :])
        return carry

    lax.fori_loop(0, NCHUNK, chunk_body, 0)


@jax.jit
def kernel(y_pred):
    bt, nb, _ = y_pred.shape
    x = y_pred.reshape(bt * nb, C_IN)
    out = _decode_sc(x)
    return out.reshape(bt, nb, C_OUT)


# speedup vs baseline: 1.3685x; 1.2301x over previous
import functools

import jax
import jax.numpy as jnp
from jax import lax
from jax.experimental import pallas as pl
from jax.experimental.pallas import tpu as pltpu
from jax.experimental.pallas import tpu_sc as plsc

IMG = 512.0
C_IN = 39
C_OUT = 31
N = 640000
NW = 32
PER_W = N // NW
CHUNK = 80
NCHUNK = PER_W // CHUNK
GROUPS = CHUNK // 16

_mesh = plsc.VectorSubcoreMesh(core_axis_name="c", subcore_axis_name="s")


@functools.partial(
    pl.kernel,
    mesh=_mesh,
    out_type=jax.ShapeDtypeStruct((N, C_OUT), jnp.float32),
    scratch_types=[
        pltpu.VMEM((2, CHUNK, C_IN), jnp.float32),
        pltpu.VMEM((2, CHUNK, C_OUT), jnp.float32),
        pltpu.SemaphoreType.DMA,
        pltpu.SemaphoreType.DMA,
        pltpu.SemaphoreType.DMA,
        pltpu.SemaphoreType.DMA,
    ],
    compiler_params=pltpu.CompilerParams(
        needs_layout_passes=False, use_tc_tiling_on_sc=True
    ),
)
def _decode_sc(x_hbm, o_hbm, in_v, out_v, in_sem0, in_sem1, wb_sem0, wb_sem1):
    cid = lax.axis_index("c")
    sid = lax.axis_index("s")
    wid = sid * 2 + cid
    base = wid * PER_W
    lanes = lax.iota(jnp.int32, 16)
    in_sems = (in_sem0, in_sem1)
    wb_sems = (wb_sem0, wb_sem1)

    def in_copy(ci, b):
        cbase = base + ci * CHUNK
        return pltpu.make_async_copy(
            x_hbm.at[pl.ds(cbase, CHUNK), :], in_v.at[b], in_sems[b]
        )

    def wb_copy(ci, b):
        cbase = base + ci * CHUNK
        return pltpu.make_async_copy(
            out_v.at[b], o_hbm.at[pl.ds(cbase, CHUNK), :], wb_sems[b]
        )

    def compute(b):
        iv = in_v.at[b]
        ov = out_v.at[b]
        for g in range(GROUPS):
            rows = g * 16 + lanes

            def gat(c):
                col = jnp.full((16,), c, jnp.int32)
                return plsc.load_gather(iv, [rows, col])

            def scat(c, val):
                col = jnp.full((16,), c, jnp.int32)
                plsc.store_scatter(ov, [rows, col], val)

            cx = gat(31)
            cy = gat(32)
            vxw = gat(35) * gat(33)
            vyh = gat(36) * gat(34)
            for i in range(5):
                kx = (gat(21 + 2 * i) * vxw + cx) * IMG
                ky = (gat(22 + 2 * i) * vyh + cy) * IMG
                scat(21 + 2 * i, kx)
                scat(22 + 2 * i, ky)
            for c in range(21):
                scat(c, gat(c))

    in_copy(0, 0).start()
    in_copy(1, 1).start()

    def pair_body(i, carry):
        for b in range(2):
            ci = i * 2 + b

            @pl.when(i > 0)
            def _():
                wb_copy(ci - 2, b).wait()

            in_copy(ci, b).wait()
            compute(b)
            wb_copy(ci, b).start()

            @pl.when(ci + 2 < NCHUNK)
            def _():
                in_copy(ci + 2, b).start()

        return carry

    lax.fori_loop(0, NCHUNK // 2, pair_body, 0)
    wb_copy(NCHUNK - 2, 0).wait()
    wb_copy(NCHUNK - 1, 1).wait()


@jax.jit
def kernel(y_pred):
    bt, nb, _ = y_pred.shape
    x = y_pred.reshape(bt * nb, C_IN)
    out = _decode_sc(x)
    return out.reshape(bt, nb, C_OUT)


# speedup vs baseline: 1.9796x; 1.4466x over previous
import jax
import jax.numpy as jnp
from jax import lax
from jax.experimental import pallas as pl
from jax.experimental.pallas import tpu as pltpu

IMG = 512.0
C_IN = 39
C_OUT = 31
N = 640000
BLK = 5120
S = N // BLK
D = 6


def _decode(x):
    b = x.shape[0]
    t = x * jnp.roll(x, -2, axis=1)
    col = lax.broadcasted_iota(jnp.int32, (b, C_OUT), 1)
    is_kp = col >= 21
    odd = (col % 2) == 1
    scale = jnp.take_along_axis(t, jnp.where(odd, 33, 34), axis=1)
    shift = jnp.take_along_axis(x, jnp.where(odd, 31, 32), axis=1)
    xo = x[:, :C_OUT]
    return jnp.where(is_kp, (xo * scale + shift) * IMG, xo)


def _body(x_hbm, o_hbm, in_b, out_b, in_sems, out_sems):
    def in_cp(s, bf):
        return pltpu.make_async_copy(
            x_hbm.at[pl.ds(s * BLK, BLK), :], in_b.at[bf], in_sems.at[bf]
        )

    def out_cp(s, bf):
        return pltpu.make_async_copy(
            out_b.at[bf], o_hbm.at[pl.ds(s * BLK, BLK), :], out_sems.at[bf]
        )

    for j in range(D):
        in_cp(j, j).start()

    def step(s, carry):
        bf = lax.rem(s, D)
        in_cp(s, bf).wait()

        @pl.when(s >= D)
        def _():
            out_cp(s - D, bf).wait()

        out_b.at[bf][...] = _decode(in_b.at[bf][...])
        out_cp(s, bf).start()

        @pl.when(s + D < S)
        def _():
            in_cp(s + D, bf).start()

        return carry

    lax.fori_loop(0, S, step, 0)
    for s in range(S - D, S):
        out_cp(s, s % D).wait()


@jax.jit
def kernel(y_pred):
    bt, nb, _ = y_pred.shape
    x = y_pred.reshape(bt * nb, C_IN)
    out = pl.pallas_call(
        _body,
        in_specs=[pl.BlockSpec(memory_space=pltpu.HBM)],
        out_specs=pl.BlockSpec(memory_space=pltpu.HBM),
        out_shape=jax.ShapeDtypeStruct((N, C_OUT), jnp.float32),
        scratch_shapes=[
            pltpu.VMEM((D, BLK, C_IN), jnp.float32),
            pltpu.VMEM((D, BLK, C_OUT), jnp.float32),
            pltpu.SemaphoreType.DMA((D,)),
            pltpu.SemaphoreType.DMA((D,)),
        ],
    )(x)
    return out.reshape(bt, nb, C_OUT)
